# baseline (speedup 1.0000x reference)
import functools

import jax
import jax.numpy as jnp
from jax import lax
from jax.experimental import pallas as pl
from jax.experimental.pallas import tpu as pltpu
from jax.experimental.pallas import tpu_sc as plsc


def _routing_sc(router_logits):
    T, E = router_logits.shape
    info = plsc.get_sparse_core_info()
    NW = info.num_cores * info.num_subcores
    rows_per = T // NW

    mesh = plsc.VectorSubcoreMesh(core_axis_name="c", subcore_axis_name="s")

    @functools.partial(
        pl.kernel,
        mesh=mesh,
        out_type=jax.ShapeDtypeStruct((T, E), jnp.float32),
        scratch_types=[
            pltpu.VMEM((rows_per, E), jnp.float32),
            pltpu.VMEM((rows_per, E), jnp.float32),
        ],
        compiler_params=pltpu.CompilerParams(needs_layout_passes=False),
    )
    def k(logits_hbm, out_hbm, in_v, out_v):
        wid = lax.axis_index("s") * info.num_cores + lax.axis_index("c")
        base = wid * rows_per
        pltpu.sync_copy(logits_hbm.at[pl.ds(base, rows_per)], in_v)
        iot = lax.iota(jnp.int32, 16)
        for t in range(rows_per):
            v = in_v[t]
            m = jnp.max(v)
            ex = jnp.exp(v - m)
            m1 = jnp.max(ex)
            i1 = jnp.min(jnp.where(ex == m1, iot, E))
            ex2 = jnp.where(iot == i1, jnp.float32(-1.0), ex)
            m2 = jnp.max(ex2)
            i2 = jnp.min(jnp.where(ex2 == m2, iot, E))
            num = jnp.where(
                iot == i1, m1, jnp.where(iot == i2, m2, jnp.float32(0.0))
            )
            den = jnp.broadcast_to(m1 + m2, (16,))
            out_v[t] = num / den
        pltpu.sync_copy(out_v, out_hbm.at[pl.ds(base, rows_per)])

    return k(router_logits)


def _moe_tc(x, full_w, gate_w, up_w, down_w):
    T, H = x.shape
    E, I, _ = gate_w.shape
    TI = 704
    NI = I // TI

    dw4 = down_w.reshape(E, H, NI, TI)

    def body(x_ref, w_ref, g_ref, u_ref, d_ref, o_ref, h_ref):
        e = pl.program_id(0)
        i = pl.program_id(1)

        @pl.when((e == 0) & (i == 0))
        def _init():
            o_ref[...] = jnp.zeros_like(o_ref)

        xv = x_ref[...]
        g = lax.dot_general(
            xv, g_ref[0], (((1,), (1,)), ((), ())),
            preferred_element_type=jnp.float32,
        )
        u = lax.dot_general(
            xv, u_ref[0], (((1,), (1,)), ((), ())),
            preferred_element_type=jnp.float32,
        )
        h_ref[i] = g * jax.nn.sigmoid(g) * u

        @pl.when(i == NI - 1)
        def _down():
            dt = d_ref[0]
            po = lax.dot_general(
                h_ref[0], dt[:, 0, :], (((1,), (1,)), ((), ())),
                preferred_element_type=jnp.float32,
            )
            for j in range(1, NI):
                po += lax.dot_general(
                    h_ref[j], dt[:, j, :], (((1,), (1,)), ((), ())),
                    preferred_element_type=jnp.float32,
                )
            lane = lax.broadcasted_iota(jnp.int32, (T, E), 1)
            wcol = jnp.sum(
                jnp.where(lane == e, w_ref[...], jnp.float32(0.0)),
                axis=1,
                keepdims=True,
            )
            o_ref[...] += wcol * po

    return pl.pallas_call(
        body,
        grid=(E, NI),
        in_specs=[
            pl.BlockSpec((T, H), lambda e, i: (0, 0)),
            pl.BlockSpec((T, E), lambda e, i: (0, 0)),
            pl.BlockSpec((1, TI, H), lambda e, i: (e, i, 0)),
            pl.BlockSpec((1, TI, H), lambda e, i: (e, i, 0)),
            pl.BlockSpec((1, H, NI, TI), lambda e, i: (e, 0, 0, 0)),
        ],
        out_specs=pl.BlockSpec((T, H), lambda e, i: (0, 0)),
        out_shape=jax.ShapeDtypeStruct((T, H), jnp.float32),
        scratch_shapes=[pltpu.VMEM((NI, T, TI), jnp.float32)],
        compiler_params=pltpu.CompilerParams(
            dimension_semantics=("arbitrary", "arbitrary"),
        ),
    )(x, full_w, gate_w, up_w, dw4)


@jax.jit
def kernel(x, router_logits, gate_w, up_w, down_w):
    full_w = _routing_sc(router_logits)
    return _moe_tc(x, full_w, gate_w, up_w, down_w)


# speedup vs baseline: 2.4222x; 2.4222x over previous
import functools

import jax
import jax.numpy as jnp
from jax import lax
from jax.experimental import pallas as pl
from jax.experimental.pallas import tpu as pltpu
from jax.experimental.pallas import tpu_sc as plsc


def _routing_sc(router_logits):
    T, E = router_logits.shape
    info = plsc.get_sparse_core_info()
    NW = info.num_cores * info.num_subcores
    rows_per = T // NW

    mesh = plsc.VectorSubcoreMesh(core_axis_name="c", subcore_axis_name="s")

    @functools.partial(
        pl.kernel,
        mesh=mesh,
        out_type=jax.ShapeDtypeStruct((T, E), jnp.float32),
        scratch_types=[
            pltpu.VMEM((rows_per, E), jnp.float32),
            pltpu.VMEM((rows_per, E), jnp.float32),
        ],
        compiler_params=pltpu.CompilerParams(needs_layout_passes=False),
    )
    def k(logits_hbm, out_hbm, in_v, out_v):
        wid = lax.axis_index("s") * info.num_cores + lax.axis_index("c")
        base = wid * rows_per
        pltpu.sync_copy(logits_hbm.at[pl.ds(base, rows_per)], in_v)
        iot = lax.iota(jnp.int32, 16)
        for t in range(rows_per):
            v = in_v[t]
            m = jnp.max(v)
            ex = jnp.exp(v - m)
            m1 = jnp.max(ex)
            i1 = jnp.min(jnp.where(ex == m1, iot, E))
            ex2 = jnp.where(iot == i1, jnp.float32(-1.0), ex)
            m2 = jnp.max(ex2)
            i2 = jnp.min(jnp.where(ex2 == m2, iot, E))
            num = jnp.where(
                iot == i1, m1, jnp.where(iot == i2, m2, jnp.float32(0.0))
            )
            den = jnp.broadcast_to(m1 + m2, (16,))
            out_v[t] = num / den
        pltpu.sync_copy(out_v, out_hbm.at[pl.ds(base, rows_per)])

    return k(router_logits)


def _moe_tc(x, full_w, gate_w, up_w, down_w):
    T, H = x.shape
    E, I, _ = gate_w.shape
    TI = 128
    NI = I // TI

    def body(x_ref, w_ref, g_ref, u_ref, d_ref, o_ref, h_ref):
        e = pl.program_id(0)
        i = pl.program_id(1)

        @pl.when((e == 0) & (i == 0))
        def _init():
            o_ref[...] = jnp.zeros_like(o_ref)

        xv = x_ref[...]
        g = lax.dot_general(
            xv, g_ref[0], (((1,), (1,)), ((), ())),
            preferred_element_type=jnp.float32,
        )
        u = lax.dot_general(
            xv, u_ref[0], (((1,), (1,)), ((), ())),
            preferred_element_type=jnp.float32,
        )
        off = pl.multiple_of(i * TI, TI)
        h_ref[:, pl.ds(off, TI)] = g * jax.nn.sigmoid(g) * u

        @pl.when(i == NI - 1)
        def _down():
            po = lax.dot_general(
                h_ref[...], d_ref[0], (((1,), (1,)), ((), ())),
                preferred_element_type=jnp.float32,
            )
            lane = lax.broadcasted_iota(jnp.int32, (T, E), 1)
            wcol = jnp.sum(
                jnp.where(lane == e, w_ref[...], jnp.float32(0.0)),
                axis=1,
                keepdims=True,
            )
            o_ref[...] += wcol * po

    return pl.pallas_call(
        body,
        grid=(E, NI),
        in_specs=[
            pl.BlockSpec((T, H), lambda e, i: (0, 0)),
            pl.BlockSpec((T, E), lambda e, i: (0, 0)),
            pl.BlockSpec((1, TI, H), lambda e, i: (e, i, 0)),
            pl.BlockSpec((1, TI, H), lambda e, i: (e, i, 0)),
            pl.BlockSpec((1, H, I), lambda e, i: (e, 0, 0)),
        ],
        out_specs=pl.BlockSpec((T, H), lambda e, i: (0, 0)),
        out_shape=jax.ShapeDtypeStruct((T, H), jnp.float32),
        scratch_shapes=[pltpu.VMEM((T, I), jnp.float32)],
        compiler_params=pltpu.CompilerParams(
            dimension_semantics=("arbitrary", "arbitrary"),
        ),
    )(x, full_w, gate_w, up_w, down_w)


@jax.jit
def kernel(x, router_logits, gate_w, up_w, down_w):
    full_w = _routing_sc(router_logits)
    return _moe_tc(x, full_w, gate_w, up_w, down_w)


# speedup vs baseline: 2.4956x; 1.0303x over previous
import functools

import jax
import jax.numpy as jnp
from jax import lax
from jax.experimental import pallas as pl
from jax.experimental.pallas import tpu as pltpu
from jax.experimental.pallas import tpu_sc as plsc


def _routing_sc(router_logits):
    T, E = router_logits.shape
    info = plsc.get_sparse_core_info()
    NW = info.num_cores * info.num_subcores
    rows_per = T // NW

    mesh = plsc.VectorSubcoreMesh(core_axis_name="c", subcore_axis_name="s")

    @functools.partial(
        pl.kernel,
        mesh=mesh,
        out_type=jax.ShapeDtypeStruct((T, E), jnp.float32),
        scratch_types=[
            pltpu.VMEM((rows_per, E), jnp.float32),
            pltpu.VMEM((rows_per, E), jnp.float32),
        ],
        compiler_params=pltpu.CompilerParams(needs_layout_passes=False),
    )
    def k(logits_hbm, out_hbm, in_v, out_v):
        wid = lax.axis_index("s") * info.num_cores + lax.axis_index("c")
        base = wid * rows_per
        pltpu.sync_copy(logits_hbm.at[pl.ds(base, rows_per)], in_v)
        iot = lax.iota(jnp.int32, 16)
        for t in range(rows_per):
            v = in_v[t]
            m = jnp.max(v)
            ex = jnp.exp(v - m)
            m1 = jnp.max(ex)
            i1 = jnp.min(jnp.where(ex == m1, iot, E))
            ex2 = jnp.where(iot == i1, jnp.float32(-1.0), ex)
            m2 = jnp.max(ex2)
            i2 = jnp.min(jnp.where(ex2 == m2, iot, E))
            num = jnp.where(
                iot == i1, m1, jnp.where(iot == i2, m2, jnp.float32(0.0))
            )
            den = jnp.broadcast_to(m1 + m2, (16,))
            out_v[t] = num / den
        pltpu.sync_copy(out_v, out_hbm.at[pl.ds(base, rows_per)])

    return k(router_logits)


def _moe_tc(x, full_w, gate_w, up_w, down_w):
    T, H = x.shape
    E, I, _ = gate_w.shape
    ND = 2
    HD = H // ND
    down_r = down_w.reshape(E, ND, HD, I)

    def body(x_ref, w_ref, g_ref, u_ref, d_ref, o_ref, h_ref):
        e = pl.program_id(0)
        i = pl.program_id(1)

        @pl.when((e == 0) & (i == 0))
        def _init():
            o_ref[...] = jnp.zeros_like(o_ref)

        @pl.when(i == 0)
        def _gu():
            xv = x_ref[...]
            g = lax.dot_general(
                xv, g_ref[0], (((1,), (1,)), ((), ())),
                preferred_element_type=jnp.float32,
            )
            u = lax.dot_general(
                xv, u_ref[0], (((1,), (1,)), ((), ())),
                preferred_element_type=jnp.float32,
            )
            h_ref[...] = g * jax.nn.sigmoid(g) * u

        po = lax.dot_general(
            h_ref[...], d_ref[0, 0], (((1,), (1,)), ((), ())),
            preferred_element_type=jnp.float32,
        )
        lane = lax.broadcasted_iota(jnp.int32, (T, E), 1)
        wcol = jnp.sum(
            jnp.where(lane == e, w_ref[...], jnp.float32(0.0)),
            axis=1,
            keepdims=True,
        )
        hoff = pl.multiple_of(i * HD, HD)
        o_ref[:, pl.ds(hoff, HD)] += wcol * po

    return pl.pallas_call(
        body,
        grid=(E, ND),
        in_specs=[
            pl.BlockSpec((T, H), lambda e, i: (0, 0)),
            pl.BlockSpec((T, E), lambda e, i: (0, 0)),
            pl.BlockSpec((1, I, H), lambda e, i: (e, 0, 0)),
            pl.BlockSpec((1, I, H), lambda e, i: (e, 0, 0)),
            pl.BlockSpec((1, 1, HD, I), lambda e, i: (e, i, 0, 0)),
        ],
        out_specs=pl.BlockSpec((T, H), lambda e, i: (0, 0)),
        out_shape=jax.ShapeDtypeStruct((T, H), jnp.float32),
        scratch_shapes=[pltpu.VMEM((T, I), jnp.float32)],
        compiler_params=pltpu.CompilerParams(
            dimension_semantics=("arbitrary", "arbitrary"),
        ),
    )(x, full_w, gate_w, up_w, down_r)


@jax.jit
def kernel(x, router_logits, gate_w, up_w, down_w):
    full_w = _routing_sc(router_logits)
    return _moe_tc(x, full_w, gate_w, up_w, down_w)


# speedup vs baseline: 2.5144x; 1.0075x over previous
import functools

import jax
import jax.numpy as jnp
from jax import lax
from jax.experimental import pallas as pl
from jax.experimental.pallas import tpu as pltpu
from jax.experimental.pallas import tpu_sc as plsc


def _routing_sc(router_logits):
    T, E = router_logits.shape
    info = plsc.get_sparse_core_info()
    NW = info.num_cores * info.num_subcores
    rows_per = T // NW

    mesh = plsc.VectorSubcoreMesh(core_axis_name="c", subcore_axis_name="s")

    @functools.partial(
        pl.kernel,
        mesh=mesh,
        out_type=jax.ShapeDtypeStruct((T, E), jnp.float32),
        scratch_types=[
            pltpu.VMEM((rows_per, E), jnp.float32),
            pltpu.VMEM((rows_per, E), jnp.float32),
        ],
        compiler_params=pltpu.CompilerParams(needs_layout_passes=False),
    )
    def k(logits_hbm, out_hbm, in_v, out_v):
        wid = lax.axis_index("s") * info.num_cores + lax.axis_index("c")
        base = wid * rows_per
        pltpu.sync_copy(logits_hbm.at[pl.ds(base, rows_per)], in_v)
        iot = lax.iota(jnp.int32, 16)
        for t in range(rows_per):
            v = in_v[t]
            m = jnp.max(v)
            ex = jnp.exp(v - m)
            m1 = jnp.max(ex)
            i1 = jnp.min(jnp.where(ex == m1, iot, E))
            ex2 = jnp.where(iot == i1, jnp.float32(-1.0), ex)
            m2 = jnp.max(ex2)
            i2 = jnp.min(jnp.where(ex2 == m2, iot, E))
            num = jnp.where(
                iot == i1, m1, jnp.where(iot == i2, m2, jnp.float32(0.0))
            )
            den = jnp.broadcast_to(m1 + m2, (16,))
            out_v[t] = num / den
        pltpu.sync_copy(out_v, out_hbm.at[pl.ds(base, rows_per)])

    return k(router_logits)


def _moe_tc(x, full_w, gate_w, up_w, down_w):
    T, H = x.shape
    E, I, _ = gate_w.shape
    ND = 2
    HD = H // ND
    HQ = HD // 2
    HH = H // 2
    down_r = down_w.reshape(E, ND, 2, HQ, I)

    def body(x_ref, w_ref, ga_ref, gb_ref, ua_ref, ub_ref, da_ref, db_ref,
             o_ref, h_ref):
        e = pl.program_id(0)
        i = pl.program_id(1)

        @pl.when((e == 0) & (i == 0))
        def _init():
            o_ref[...] = jnp.zeros_like(o_ref)

        @pl.when(i == 0)
        def _gu():
            xa = x_ref[:, 0:HH]
            xb = x_ref[:, HH:H]
            g = lax.dot_general(
                xa, ga_ref[0], (((1,), (1,)), ((), ())),
                preferred_element_type=jnp.float32,
            ) + lax.dot_general(
                xb, gb_ref[0], (((1,), (1,)), ((), ())),
                preferred_element_type=jnp.float32,
            )
            u = lax.dot_general(
                xa, ua_ref[0], (((1,), (1,)), ((), ())),
                preferred_element_type=jnp.float32,
            ) + lax.dot_general(
                xb, ub_ref[0], (((1,), (1,)), ((), ())),
                preferred_element_type=jnp.float32,
            )
            h_ref[...] = g * jax.nn.sigmoid(g) * u

        hv = h_ref[...]
        poa = lax.dot_general(
            hv, da_ref[0, 0, 0], (((1,), (1,)), ((), ())),
            preferred_element_type=jnp.float32,
        )
        pob = lax.dot_general(
            hv, db_ref[0, 0, 0], (((1,), (1,)), ((), ())),
            preferred_element_type=jnp.float32,
        )
        lane = lax.broadcasted_iota(jnp.int32, (T, E), 1)
        wcol = jnp.sum(
            jnp.where(lane == e, w_ref[...], jnp.float32(0.0)),
            axis=1,
            keepdims=True,
        )
        hoff = pl.multiple_of(i * HD, HD)
        o_ref[:, pl.ds(hoff, HQ)] += wcol * poa
        hoff2 = pl.multiple_of(i * HD + HQ, HQ)
        o_ref[:, pl.ds(hoff2, HQ)] += wcol * pob

    return pl.pallas_call(
        body,
        grid=(E, ND),
        in_specs=[
            pl.BlockSpec((T, H), lambda e, i: (0, 0)),
            pl.BlockSpec((T, E), lambda e, i: (0, 0)),
            pl.BlockSpec((1, I, HH), lambda e, i: (e, 0, 0)),
            pl.BlockSpec((1, I, HH), lambda e, i: (e, 0, 1)),
            pl.BlockSpec((1, I, HH), lambda e, i: (e, 0, 0)),
            pl.BlockSpec((1, I, HH), lambda e, i: (e, 0, 1)),
            pl.BlockSpec((1, 1, 1, HQ, I), lambda e, i: (e, i, 0, 0, 0)),
            pl.BlockSpec((1, 1, 1, HQ, I), lambda e, i: (e, i, 1, 0, 0)),
        ],
        out_specs=pl.BlockSpec((T, H), lambda e, i: (0, 0)),
        out_shape=jax.ShapeDtypeStruct((T, H), jnp.float32),
        scratch_shapes=[pltpu.VMEM((T, I), jnp.float32)],
        compiler_params=pltpu.CompilerParams(
            dimension_semantics=("arbitrary", "arbitrary"),
        ),
    )(x, full_w, gate_w, gate_w, up_w, up_w, down_r, down_r)


@jax.jit
def kernel(x, router_logits, gate_w, up_w, down_w):
    full_w = _routing_sc(router_logits)
    return _moe_tc(x, full_w, gate_w, up_w, down_w)


# speedup vs baseline: 2.6577x; 1.0570x over previous
import functools

import jax
import jax.numpy as jnp
from jax import lax
from jax.experimental import pallas as pl
from jax.experimental.pallas import tpu as pltpu
from jax.experimental.pallas import tpu_sc as plsc


def _routing_sc(router_logits):
    T, E = router_logits.shape
    info = plsc.get_sparse_core_info()
    NW = info.num_cores * info.num_subcores
    rows_per = T // NW

    mesh = plsc.VectorSubcoreMesh(core_axis_name="c", subcore_axis_name="s")

    @functools.partial(
        pl.kernel,
        mesh=mesh,
        out_type=jax.ShapeDtypeStruct((T, E), jnp.float32),
        scratch_types=[
            pltpu.VMEM((rows_per, E), jnp.float32),
            pltpu.VMEM((rows_per, E), jnp.float32),
        ],
        compiler_params=pltpu.CompilerParams(needs_layout_passes=False),
    )
    def k(logits_hbm, out_hbm, in_v, out_v):
        wid = lax.axis_index("s") * info.num_cores + lax.axis_index("c")
        base = wid * rows_per
        pltpu.sync_copy(logits_hbm.at[pl.ds(base, rows_per)], in_v)
        iot = lax.iota(jnp.int32, 16)
        for t in range(rows_per):
            v = in_v[t]
            m = jnp.max(v)
            ex = jnp.exp(v - m)
            m1 = jnp.max(ex)
            i1 = jnp.min(jnp.where(ex == m1, iot, E))
            ex2 = jnp.where(iot == i1, jnp.float32(-1.0), ex)
            m2 = jnp.max(ex2)
            i2 = jnp.min(jnp.where(ex2 == m2, iot, E))
            num = jnp.where(
                iot == i1, m1, jnp.where(iot == i2, m2, jnp.float32(0.0))
            )
            den = jnp.broadcast_to(m1 + m2, (16,))
            out_v[t] = num / den
        pltpu.sync_copy(out_v, out_hbm.at[pl.ds(base, rows_per)])

    return k(router_logits)


def _moe_tc(x, full_w, gate_w, up_w, down_w):
    T, H = x.shape
    E, I, _ = gate_w.shape
    ND = 2
    HD = H // ND
    HQ = HD // 2
    NS = 4
    IS = I // NS
    down_r = down_w.reshape(E, ND, 2, HQ, I)
    gate_r = gate_w.reshape(E, NS, IS, H)
    up_r = up_w.reshape(E, NS, IS, H)

    def body(x_ref, w_ref, g0, g1, g2, g3, u0, u1, u2, u3, da_ref, db_ref,
             o_ref, ht_ref, ot_ref):
        e = pl.program_id(0)
        i = pl.program_id(1)

        @pl.when((e == 0) & (i == 0))
        def _init():
            ot_ref[...] = jnp.zeros_like(ot_ref)

        @pl.when(i == 0)
        def _gu():
            xv = x_ref[...]
            for s, (gr, ur) in enumerate(((g0, u0), (g1, u1), (g2, u2),
                                          (g3, u3))):
                gt = lax.dot_general(
                    gr[0, 0], xv, (((1,), (1,)), ((), ())),
                    preferred_element_type=jnp.float32,
                )
                ut = lax.dot_general(
                    ur[0, 0], xv, (((1,), (1,)), ((), ())),
                    preferred_element_type=jnp.float32,
                )
                ht_ref[s * IS:(s + 1) * IS, :] = gt * jax.nn.sigmoid(gt) * ut

        htv = ht_ref[...]
        poa = lax.dot_general(
            da_ref[0, 0, 0], htv, (((1,), (0,)), ((), ())),
            preferred_element_type=jnp.float32,
        )
        pob = lax.dot_general(
            db_ref[0, 0, 0], htv, (((1,), (0,)), ((), ())),
            preferred_element_type=jnp.float32,
        )
        wt = lax.transpose(w_ref[...], (1, 0))
        sub = lax.broadcasted_iota(jnp.int32, (E, T), 0)
        wrow = jnp.sum(
            jnp.where(sub == e, wt, jnp.float32(0.0)), axis=0, keepdims=True
        )
        hoff = pl.multiple_of(i * HD, HD)
        hoff2 = pl.multiple_of(i * HD + HQ, HQ)
        ot_ref[pl.ds(hoff, HQ), :] += wrow * poa
        ot_ref[pl.ds(hoff2, HQ), :] += wrow * pob

        @pl.when((e == E - 1) & (i == ND - 1))
        def _fin():
            o_ref[...] = lax.transpose(ot_ref[...], (1, 0))

    return pl.pallas_call(
        body,
        grid=(E, ND),
        in_specs=[
            pl.BlockSpec((T, H), lambda e, i: (0, 0)),
            pl.BlockSpec((T, E), lambda e, i: (0, 0)),
            pl.BlockSpec((1, 1, IS, H), lambda e, i: (e, 0, 0, 0)),
            pl.BlockSpec((1, 1, IS, H), lambda e, i: (e, 1, 0, 0)),
            pl.BlockSpec((1, 1, IS, H), lambda e, i: (e, 2, 0, 0)),
            pl.BlockSpec((1, 1, IS, H), lambda e, i: (e, 3, 0, 0)),
            pl.BlockSpec((1, 1, IS, H), lambda e, i: (e, 0, 0, 0)),
            pl.BlockSpec((1, 1, IS, H), lambda e, i: (e, 1, 0, 0)),
            pl.BlockSpec((1, 1, IS, H), lambda e, i: (e, 2, 0, 0)),
            pl.BlockSpec((1, 1, IS, H), lambda e, i: (e, 3, 0, 0)),
            pl.BlockSpec((1, 1, 1, HQ, I), lambda e, i: (e, i, 0, 0, 0)),
            pl.BlockSpec((1, 1, 1, HQ, I), lambda e, i: (e, i, 1, 0, 0)),
        ],
        out_specs=pl.BlockSpec((T, H), lambda e, i: (0, 0)),
        out_shape=jax.ShapeDtypeStruct((T, H), jnp.float32),
        scratch_shapes=[
            pltpu.VMEM((I, T), jnp.float32),
            pltpu.VMEM((H, T), jnp.float32),
        ],
        compiler_params=pltpu.CompilerParams(
            dimension_semantics=("arbitrary", "arbitrary"),
            vmem_limit_bytes=63 * 1024 * 1024,
        ),
    )(x, full_w, gate_r, gate_r, gate_r, gate_r, up_r, up_r, up_r, up_r,
      down_r, down_r)


@jax.jit
def kernel(x, router_logits, gate_w, up_w, down_w):
    full_w = _routing_sc(router_logits)
    return _moe_tc(x, full_w, gate_w, up_w, down_w)
